# baseline (speedup 1.0000x reference)
import functools

import jax
import jax.numpy as jnp
from jax import lax
from jax.experimental import pallas as pl
from jax.experimental.pallas import tpu as pltpu
from jax.experimental.pallas import tpu_sc as plsc

HIDDEN = 32
NC = 2
NS = 16
NW = NC * NS
B = 4096
J = 200
IT = B // 128
JT = J // 8
GB = 4
NG = J // GB

_mesh = plsc.VectorSubcoreMesh(core_axis_name="c", subcore_axis_name="s")


def _transpose_block(rows_v, rowsT_v, g, b):
    ig = jnp.full((16,), g, dtype=jnp.int32)
    ib = jnp.full((16,), b, dtype=jnp.int32)
    lane = lax.iota(jnp.int32, 16)
    for c in range(HIDDEN):
        ic = jnp.full((16,), c, dtype=jnp.int32)
        for k in range(8):
            vals = plsc.load_gather(rows_v, [ig, ib, lane + k * 16, ic])
            rowsT_v[g, b, c // 8, c % 8, pl.ds(k * 16, 16)] = vals


@functools.partial(
    pl.kernel,
    mesh=_mesh,
    compiler_params=pltpu.CompilerParams(
        use_tc_tiling_on_sc=False, needs_layout_passes=False
    ),
    out_type=jax.ShapeDtypeStruct((J, 4, IT, 8, 128), jnp.float32),
    scratch_types=[
        pltpu.VMEM((JT, 8, 128), jnp.int32),
        pltpu.VMEM((2, GB, 128, HIDDEN), jnp.float32),
        pltpu.VMEM((2, GB, 4, 8, 128), jnp.float32),
        pltpu.SemaphoreType.DMA,
        pltpu.SemaphoreType.DMA,
    ],
)
def _gather_kernel(idx_hbm, table_hbm, out_hbm, idx_v, rows_v, rowsT_v, gsem, osem):
    w = lax.axis_index("s") * NC + lax.axis_index("c")
    pltpu.sync_copy(idx_hbm.at[:, w], idx_v)

    def issue_gathers(m, g):
        jt = lax.div(m, 2)
        jl0 = lax.rem(m, 2) * GB
        for b in range(GB):
            pltpu.async_copy(
                table_hbm.at[idx_v.at[jt, jl0 + b]], rows_v.at[g, b], gsem
            )

    def drain_gathers(m, g):
        jt = lax.div(m, 2)
        jl0 = lax.rem(m, 2) * GB
        for b in range(GB):
            pltpu.make_async_copy(
                table_hbm.at[idx_v.at[jt, jl0 + b]], rows_v.at[g, b], gsem
            ).wait()

    def issue_out(m, g):
        j0 = lax.div(m, 2) * 8 + lax.rem(m, 2) * GB
        pltpu.async_copy(
            rowsT_v.at[g], out_hbm.at[pl.ds(j0, GB), :, w], osem
        )

    def drain_out(g):
        pltpu.make_async_copy(
            rowsT_v.at[g], out_hbm.at[pl.ds(0, GB), :, w], osem
        ).wait()

    issue_gathers(0, 0)

    @pl.loop(0, NG // 2)
    def _(n):
        m0 = n * 2

        issue_gathers(m0 + 1, 1)
        drain_gathers(m0, 0)

        @pl.when(n > 0)
        def _():
            drain_out(0)

        for b in range(GB):
            _transpose_block(rows_v, rowsT_v, 0, b)
        issue_out(m0, 0)

        @pl.when(n + 1 < NG // 2)
        def _():
            issue_gathers(m0 + 2, 0)

        drain_gathers(m0 + 1, 1)

        @pl.when(n > 0)
        def _():
            drain_out(1)

        for b in range(GB):
            _transpose_block(rows_v, rowsT_v, 1, b)
        issue_out(m0 + 1, 1)

    drain_out(0)
    drain_out(1)


def kernel(obs, obs_embedding_weight):
    idx4 = (
        obs.astype(jnp.int32)
        .T.reshape(JT, 8, IT, 128)
        .transpose(0, 2, 1, 3)
    )
    out5 = _gather_kernel(idx4, obs_embedding_weight)
    return out5.transpose(2, 4, 0, 1, 3).reshape(B, J, HIDDEN)


# speedup vs baseline: 1.5998x; 1.5998x over previous
import functools

import jax
import jax.numpy as jnp
from jax import lax
from jax.experimental import pallas as pl
from jax.experimental.pallas import tpu as pltpu
from jax.experimental.pallas import tpu_sc as plsc

HIDDEN = 32
NC = 2
NS = 16
NW = NC * NS
B = 4096
J = 200
IT = B // 128
JT = J // 8
GB = 4
NG = J // GB
TPAD = 129

_mesh = plsc.VectorSubcoreMesh(core_axis_name="c", subcore_axis_name="s")


def _transpose_block(rows_v, rowsT_v, g, b):
    lane = lax.iota(jnp.int32, 16)
    ig = jnp.full((16,), g, dtype=jnp.int32)
    ib = jnp.full((16,), b, dtype=jnp.int32)
    c_lo = lane
    c_hi = lane + 16
    for i0 in range(0, 128, 8):
        vals = []
        for r in range(8):
            vals.append(rows_v[g, b, i0 + r, pl.ds(0, 16)])
            vals.append(rows_v[g, b, i0 + r, pl.ds(16, 16)])
        for r in range(8):
            ii = jnp.full((16,), i0 + r, dtype=jnp.int32)
            plsc.store_scatter(rowsT_v, [ig, ib, c_lo, ii], vals[2 * r])
            plsc.store_scatter(rowsT_v, [ig, ib, c_hi, ii], vals[2 * r + 1])


@functools.partial(
    pl.kernel,
    mesh=_mesh,
    compiler_params=pltpu.CompilerParams(
        use_tc_tiling_on_sc=False, needs_layout_passes=False
    ),
    out_type=jax.ShapeDtypeStruct((J, 4, IT, 8, 128), jnp.float32),
    scratch_types=[
        pltpu.VMEM((JT, 8, 128), jnp.int32),
        pltpu.VMEM((2, GB, 128, HIDDEN), jnp.float32),
        pltpu.VMEM((2, GB, HIDDEN, TPAD), jnp.float32),
        pltpu.SemaphoreType.DMA,
        pltpu.SemaphoreType.DMA,
    ],
)
def _gather_kernel(idx_hbm, table_hbm, out_hbm, idx_v, rows_v, rowsT_v, gsem, osem):
    w = lax.axis_index("s") * NC + lax.axis_index("c")
    pltpu.sync_copy(idx_hbm.at[:, w], idx_v)

    def issue_gathers(m, g):
        jt = lax.div(m, 2)
        jl0 = lax.rem(m, 2) * GB
        for b in range(GB):
            pltpu.async_copy(
                table_hbm.at[idx_v.at[jt, jl0 + b]], rows_v.at[g, b], gsem
            )

    def drain_gathers(m, g):
        jt = lax.div(m, 2)
        jl0 = lax.rem(m, 2) * GB
        for b in range(GB):
            pltpu.make_async_copy(
                table_hbm.at[idx_v.at[jt, jl0 + b]], rows_v.at[g, b], gsem
            ).wait()

    def issue_out(m, g):
        j0 = lax.div(m, 2) * 8 + lax.rem(m, 2) * GB
        for ct in range(4):
            pltpu.async_copy(
                rowsT_v.at[g, :, pl.ds(ct * 8, 8), pl.ds(0, 128)],
                out_hbm.at[pl.ds(j0, GB), ct, w],
                osem,
            )

    def drain_out(g):
        for ct in range(4):
            pltpu.make_async_copy(
                rowsT_v.at[g, :, pl.ds(ct * 8, 8), pl.ds(0, 128)],
                out_hbm.at[pl.ds(0, GB), ct, w],
                osem,
            ).wait()

    issue_gathers(0, 0)

    @pl.loop(0, NG // 2)
    def _(n):
        m0 = n * 2

        issue_gathers(m0 + 1, 1)
        drain_gathers(m0, 0)

        @pl.when(n > 0)
        def _():
            drain_out(0)

        for b in range(GB):
            _transpose_block(rows_v, rowsT_v, 0, b)
        issue_out(m0, 0)

        @pl.when(n + 1 < NG // 2)
        def _():
            issue_gathers(m0 + 2, 0)

        drain_gathers(m0 + 1, 1)

        @pl.when(n > 0)
        def _():
            drain_out(1)

        for b in range(GB):
            _transpose_block(rows_v, rowsT_v, 1, b)
        issue_out(m0 + 1, 1)

    drain_out(0)
    drain_out(1)


def kernel(obs, obs_embedding_weight):
    idx4 = (
        obs.astype(jnp.int32)
        .T.reshape(JT, 8, IT, 128)
        .transpose(0, 2, 1, 3)
    )
    out5 = _gather_kernel(idx4, obs_embedding_weight)
    return out5.transpose(2, 4, 0, 1, 3).reshape(B, J, HIDDEN)
